# baseline (speedup 1.0000x reference)
import functools

import jax
import jax.numpy as jnp
from jax import lax
from jax.experimental import pallas as pl
from jax.experimental.pallas import tpu as pltpu
from jax.experimental.pallas import tpu_sc as plsc

_B = 2048
_N = 128
_TOTAL = _N * _N
_BUFW = _TOTAL + (_N - 1) * _N


def _sc_shift_rows(x_flat, ports_i32):
    info = plsc.get_sparse_core_info()
    nw = info.num_cores * info.num_subcores
    rows_per_w = _B // nw

    mesh = plsc.VectorSubcoreMesh(core_axis_name="c", subcore_axis_name="s")

    @functools.partial(
        pl.kernel,
        mesh=mesh,
        out_type=jax.ShapeDtypeStruct((_B, _TOTAL), jnp.float32),
        scratch_types=[
            pltpu.VMEM((rows_per_w,), jnp.int32),
            pltpu.VMEM((_BUFW,), jnp.float32),
            pltpu.VMEM((_BUFW,), jnp.float32),
            pltpu.VMEM((_BUFW,), jnp.float32),
            pltpu.VMEM((_BUFW,), jnp.float32),
            pltpu.SemaphoreType.DMA,
            pltpu.SemaphoreType.DMA,
            pltpu.SemaphoreType.DMA,
            pltpu.SemaphoreType.DMA,
            pltpu.SemaphoreType.DMA,
            pltpu.SemaphoreType.DMA,
            pltpu.SemaphoreType.DMA,
            pltpu.SemaphoreType.DMA,
        ],
    )
    def body(x_hbm, ports_hbm, out_hbm, ports_v,
             bufa, bufb, bufc, bufd,
             rda, rdb, rdc, rdd, wra, wrb, wrc, wrd):
        wid = lax.axis_index("s") * info.num_cores + lax.axis_index("c")
        base = wid * rows_per_w

        pltpu.sync_copy(ports_hbm.at[pl.ds(base, rows_per_w)], ports_v)

        bufs = (bufa, bufb, bufc, bufd)
        rd_sems = (rda, rdb, rdc, rdd)
        wr_sems = (wra, wrb, wrc, wrd)

        n_chunks = 8
        chunk = _TOTAL // n_chunks

        def port_of(i):
            grp = ports_v[pl.ds((i // 16) * 16, 16)]
            return grp[i % 16]

        def rw_read(i, b, do_start):
            p = port_of(i)
            vstart = p * _N
            for c in range(n_chunks):
                h = pltpu.make_async_copy(
                    x_hbm.at[pl.ds((base + i) * _TOTAL + c * chunk, chunk)],
                    bufs[b].at[pl.ds(c * chunk, chunk)],
                    rd_sems[b],
                )
                if c == n_chunks - 1:
                    h.start() if do_start else h.wait()
                else:
                    @pl.when((c + 1) * chunk > vstart)
                    def _(h=h):
                        h.start() if do_start else h.wait()

        def start_write(i, b):
            p = port_of(i)
            return pltpu.async_copy(
                bufs[b].at[pl.ds(p * _N, _TOTAL)],
                out_hbm.at[base + i],
                wr_sems[b],
            )

        rw_read(0, 0, True)
        rw_read(1, 1, True)

        def zero_body(t, carry):
            z = jnp.zeros((16,), jnp.float32)
            for u in range(8):
                off = _TOTAL + 128 * t + 16 * u
                bufa[pl.ds(off, 16)] = z
                bufb[pl.ds(off, 16)] = z
                bufc[pl.ds(off, 16)] = z
                bufd[pl.ds(off, 16)] = z
            return carry

        lax.fori_loop(0, (_BUFW - _TOTAL) // 128, zero_body, 0)

        h_wr = [None, None, None, None]
        for i in range(rows_per_w):
            b = i % 4
            rw_read(i, b, False)
            h_wr[b] = start_write(i, b)
            j = i + 2
            if j < rows_per_w:
                jb = j % 4
                if j >= 4:
                    h_wr[jb].wait()
                rw_read(j, jb, True)
        for h in h_wr:
            if h is not None:
                h.wait()

    return body(x_flat, ports_i32)


def kernel(x, ports):
    x_flat = x.astype(jnp.float32).reshape(_B * _TOTAL)
    return _sc_shift_rows(x_flat, ports.astype(jnp.int32))


# speedup vs baseline: 1.0094x; 1.0094x over previous
import functools

import jax
import jax.numpy as jnp
from jax import lax
from jax.experimental import pallas as pl
from jax.experimental.pallas import tpu as pltpu
from jax.experimental.pallas import tpu_sc as plsc

_B = 2048
_N = 128
_TOTAL = _N * _N
_BUFW = _TOTAL + (_N - 1) * _N


def _sc_shift_rows(x_flat, ports_i32):
    info = plsc.get_sparse_core_info()
    nw = info.num_cores * info.num_subcores
    rows_per_w = _B // nw

    mesh = plsc.VectorSubcoreMesh(core_axis_name="c", subcore_axis_name="s")

    @functools.partial(
        pl.kernel,
        mesh=mesh,
        out_type=jax.ShapeDtypeStruct((_B, _TOTAL), jnp.float32),
        scratch_types=[
            pltpu.VMEM((rows_per_w,), jnp.int32),
            pltpu.VMEM((_BUFW,), jnp.float32),
            pltpu.VMEM((_BUFW,), jnp.float32),
            pltpu.VMEM((_BUFW,), jnp.float32),
            pltpu.VMEM((_BUFW,), jnp.float32),
            pltpu.SemaphoreType.DMA,
            pltpu.SemaphoreType.DMA,
            pltpu.SemaphoreType.DMA,
            pltpu.SemaphoreType.DMA,
            pltpu.SemaphoreType.DMA,
            pltpu.SemaphoreType.DMA,
            pltpu.SemaphoreType.DMA,
            pltpu.SemaphoreType.DMA,
        ],
    )
    def body(x_hbm, ports_hbm, out_hbm, ports_v,
             bufa, bufb, bufc, bufd,
             rda, rdb, rdc, rdd, wra, wrb, wrc, wrd):
        wid = lax.axis_index("s") * info.num_cores + lax.axis_index("c")
        base = wid * rows_per_w

        pltpu.sync_copy(ports_hbm.at[pl.ds(base, rows_per_w)], ports_v)

        bufs = (bufa, bufb, bufc, bufd)
        rd_sems = (rda, rdb, rdc, rdd)
        wr_sems = (wra, wrb, wrc, wrd)

        n_chunks = 4
        chunk = _TOTAL // n_chunks

        def port_of(i):
            grp = ports_v[pl.ds((i // 16) * 16, 16)]
            return grp[i % 16]

        def rw_read(i, b, do_start):
            p = port_of(i)
            vstart = p * _N
            for c in range(n_chunks):
                h = pltpu.make_async_copy(
                    x_hbm.at[pl.ds((base + i) * _TOTAL + c * chunk, chunk)],
                    bufs[b].at[pl.ds(c * chunk, chunk)],
                    rd_sems[b],
                )
                if c == n_chunks - 1:
                    h.start() if do_start else h.wait()
                else:
                    @pl.when((c + 1) * chunk > vstart)
                    def _(h=h):
                        h.start() if do_start else h.wait()

        def start_write(i, b):
            p = port_of(i)
            return pltpu.async_copy(
                bufs[b].at[pl.ds(p * _N, _TOTAL)],
                out_hbm.at[base + i],
                wr_sems[b],
            )

        rw_read(0, 0, True)
        rw_read(1, 1, True)

        def zero_body(t, carry):
            z = jnp.zeros((16,), jnp.float32)
            for u in range(8):
                off = _TOTAL + 128 * t + 16 * u
                bufa[pl.ds(off, 16)] = z
                bufb[pl.ds(off, 16)] = z
                bufc[pl.ds(off, 16)] = z
                bufd[pl.ds(off, 16)] = z
            return carry

        lax.fori_loop(0, (_BUFW - _TOTAL) // 128, zero_body, 0)

        h_wr = [None, None, None, None]
        for i in range(rows_per_w):
            b = i % 4
            rw_read(i, b, False)
            h_wr[b] = start_write(i, b)
            j = i + 2
            if j < rows_per_w:
                jb = j % 4
                if j >= 4:
                    h_wr[jb].wait()
                rw_read(j, jb, True)
        for h in h_wr:
            if h is not None:
                h.wait()

    return body(x_flat, ports_i32)


def kernel(x, ports):
    x_flat = x.astype(jnp.float32).reshape(_B * _TOTAL)
    return _sc_shift_rows(x_flat, ports.astype(jnp.int32))


# speedup vs baseline: 1.0290x; 1.0194x over previous
import functools

import jax
import jax.numpy as jnp
from jax import lax
from jax.experimental import pallas as pl
from jax.experimental.pallas import tpu as pltpu
from jax.experimental.pallas import tpu_sc as plsc

_B = 2048
_N = 128
_TOTAL = _N * _N
_BUFW = _TOTAL + (_N - 1) * _N


def _sc_shift_rows(x_flat, ports_i32):
    info = plsc.get_sparse_core_info()
    nw = info.num_cores * info.num_subcores
    rows_per_w = _B // nw

    mesh = plsc.VectorSubcoreMesh(core_axis_name="c", subcore_axis_name="s")

    @functools.partial(
        pl.kernel,
        mesh=mesh,
        out_type=jax.ShapeDtypeStruct((_B, _TOTAL), jnp.float32),
        scratch_types=[
            pltpu.VMEM((rows_per_w,), jnp.int32),
            pltpu.VMEM((_BUFW,), jnp.float32),
            pltpu.VMEM((_BUFW,), jnp.float32),
            pltpu.VMEM((_BUFW,), jnp.float32),
            pltpu.VMEM((_BUFW,), jnp.float32),
            pltpu.SemaphoreType.DMA,
            pltpu.SemaphoreType.DMA,
            pltpu.SemaphoreType.DMA,
            pltpu.SemaphoreType.DMA,
            pltpu.SemaphoreType.DMA,
            pltpu.SemaphoreType.DMA,
            pltpu.SemaphoreType.DMA,
            pltpu.SemaphoreType.DMA,
        ],
    )
    def body(x_hbm, ports_hbm, out_hbm, ports_v,
             bufa, bufb, bufc, bufd,
             rda, rdb, rdc, rdd, wra, wrb, wrc, wrd):
        wid = lax.axis_index("s") * info.num_cores + lax.axis_index("c")
        base = wid * rows_per_w

        bufs = (bufa, bufb, bufc, bufd)
        rd_sems = (rda, rdb, rdc, rdd)
        wr_sems = (wra, wrb, wrc, wrd)

        n_chunks = 4
        chunk = _TOTAL // n_chunks

        _port_cache = {}

        def port_of(i):
            if i not in _port_cache:
                grp = ports_v[pl.ds((i // 16) * 16, 16)]
                _port_cache[i] = grp[i % 16]
            return _port_cache[i]

        def rw_read(i, b, do_start, unconditional=False):
            for c in range(n_chunks):
                h = pltpu.make_async_copy(
                    x_hbm.at[pl.ds((base + i) * _TOTAL + c * chunk, chunk)],
                    bufs[b].at[pl.ds(c * chunk, chunk)],
                    rd_sems[b],
                )
                if unconditional or c == n_chunks - 1:
                    h.start() if do_start else h.wait()
                else:
                    @pl.when((c + 1) * chunk > port_of(i) * _N)
                    def _(h=h):
                        h.start() if do_start else h.wait()

        def start_write(i, b):
            p = port_of(i)
            return pltpu.async_copy(
                bufs[b].at[pl.ds(p * _N, _TOTAL)],
                out_hbm.at[base + i],
                wr_sems[b],
            )

        rw_read(0, 0, True, unconditional=True)
        rw_read(1, 1, True, unconditional=True)

        pltpu.sync_copy(ports_hbm.at[pl.ds(base, rows_per_w)], ports_v)

        def zero_body(t, carry):
            z = jnp.zeros((16,), jnp.float32)
            for u in range(8):
                off = _TOTAL + 128 * t + 16 * u
                bufa[pl.ds(off, 16)] = z
                bufb[pl.ds(off, 16)] = z
                bufc[pl.ds(off, 16)] = z
                bufd[pl.ds(off, 16)] = z
            return carry

        lax.fori_loop(0, (_BUFW - _TOTAL) // 128, zero_body, 0)

        h_wr = [None, None, None, None]
        for i in range(rows_per_w):
            b = i % 4
            rw_read(i, b, False, unconditional=(i < 2))
            h_wr[b] = start_write(i, b)
            j = i + 2
            if j < rows_per_w:
                jb = j % 4
                if j >= 4:
                    h_wr[jb].wait()
                rw_read(j, jb, True)
        for h in h_wr:
            if h is not None:
                h.wait()

    return body(x_flat, ports_i32)


def kernel(x, ports):
    x_flat = x.astype(jnp.float32).reshape(_B * _TOTAL)
    return _sc_shift_rows(x_flat, ports.astype(jnp.int32))
